# baseline (speedup 1.0000x reference)
import functools

import jax
import jax.numpy as jnp
from jax import lax
from jax.experimental import pallas as pl
from jax.experimental.pallas import tpu as pltpu
from jax.experimental.pallas import tpu_sc as plsc

N = 10000
DEG = 32
D = 128
OUT = 128
B = 4096
S1 = 16
S2 = 4

NC, NS, L = 2, 16, 16
NW = NC * NS
NODES_PER_W = 320
CH_A = 32
NCH_A = NODES_PER_W // CH_A
NBUF_A = 5
B_PER_W = B // NW
CH_C = 8
NCH_C = B_PER_W // CH_C
NBUF_C = 4

_MESH = plsc.VectorSubcoreMesh(
    core_axis_name="c", subcore_axis_name="s", num_cores=NC, num_subcores=NS
)


@functools.partial(
    pl.kernel,
    out_type=jax.ShapeDtypeStruct((N, D), jnp.float32),
    mesh=_MESH,
    scratch_types=[
        pltpu.VMEM((NODES_PER_W * S2,), jnp.int32),
    ]
    + [pltpu.VMEM((CH_A * S2, D), jnp.float32) for _ in range(NBUF_A)]
    + [pltpu.VMEM((NODES_PER_W, D), jnp.float32)]
    + [pltpu.SemaphoreType.DMA for _ in range(NBUF_A)],
)
def _preagg(idx_hbm, feat_hbm, out_hbm, idx_v, r0, r1, r2, r3, r4, ob,
            s0, s1, s2, s3, s4):
    rows = (r0, r1, r2, r3, r4)
    sems = (s0, s1, s2, s3, s4)
    wid = lax.axis_index("s") * NC + lax.axis_index("c")
    base = jnp.minimum(wid * NODES_PER_W, N - NODES_PER_W)
    pltpu.sync_copy(idx_hbm.at[pl.ds(base * S2, NODES_PER_W * S2)], idx_v)

    def issue(ch):
        b = ch % NBUF_A
        src = feat_hbm.at[idx_v.at[pl.ds(ch * CH_A * S2, CH_A * S2)]]
        return pltpu.async_copy(src, rows[b], sems[b])

    cps = {ch: issue(ch) for ch in range(NBUF_A)}
    for ch in range(NCH_A):
        b = ch % NBUF_A
        cps[ch].wait()

        @plsc.parallel_loop(0, CH_A, unroll=2)
        def node(i, _rv=rows[b], _off=ch * CH_A):
            for k in range(D // L):
                s = _rv[i * S2, pl.ds(k * L, L)]
                for p in range(1, S2):
                    s = s + _rv[i * S2 + p, pl.ds(k * L, L)]
                ob[_off + i, pl.ds(k * L, L)] = s * (1.0 / S2)
        if ch + NBUF_A < NCH_A:
            cps[ch + NBUF_A] = issue(ch + NBUF_A)
    pltpu.sync_copy(ob, out_hbm.at[pl.ds(base, NODES_PER_W)])


@functools.partial(
    pl.kernel,
    out_type=jax.ShapeDtypeStruct((B, OUT), jnp.float32),
    mesh=_MESH,
    scratch_types=[
        pltpu.VMEM((B_PER_W,), jnp.int32),
        pltpu.VMEM((B_PER_W, 128), jnp.int32),
        pltpu.VMEM((B_PER_W, OUT), jnp.float32),
    ]
    + [pltpu.VMEM((CH_C * S1,), jnp.int32) for _ in range(NBUF_C)]
    + [pltpu.VMEM((CH_C * S1, OUT), jnp.float32) for _ in range(NBUF_C)]
    + [pltpu.VMEM((B_PER_W, OUT), jnp.float32)]
    + [pltpu.SemaphoreType.DMA for _ in range(NBUF_C + 2)],
)
def _batch(nb_hbm, adj_hbm, gs_hbm, gn_hbm, out_hbm,
           nb_v, adjr_v, gself_v, i0, i1, i2, i3, r0, r1, r2, r3, outb,
           s0, s1, s2, s3, sa, sh):
    nidx = (i0, i1, i2, i3)
    nrows = (r0, r1, r2, r3)
    sems = (s0, s1, s2, s3)
    wid = lax.axis_index("s") * NC + lax.axis_index("c")
    base = wid * B_PER_W
    pltpu.sync_copy(nb_hbm.at[pl.ds(base, B_PER_W)], nb_v)
    cp_adj = pltpu.async_copy(adj_hbm.at[nb_v], adjr_v, sa)
    cp_self = pltpu.async_copy(gs_hbm.at[nb_v], gself_v, sh)
    cp_adj.wait()

    def issue(ch):
        b = ch % NBUF_C
        _ni = nidx[b]

        @plsc.parallel_loop(0, CH_C, unroll=2)
        def build(j, _off=ch * CH_C):
            _ni[pl.ds(j * S1, S1)] = adjr_v[_off + j, pl.ds(0, S1)]
        return pltpu.async_copy(gn_hbm.at[_ni], nrows[b], sems[b])

    cps = {ch: issue(ch) for ch in range(NBUF_C)}
    cp_self.wait()
    for ch in range(NCH_C):
        b = ch % NBUF_C
        cps[ch].wait()

        @plsc.parallel_loop(0, CH_C)
        def bacc(j, _rv=nrows[b], _off=ch * CH_C):
            @plsc.parallel_loop(0, OUT // L, unroll=2)
            def kstep(k):
                s = _rv[j * S1, pl.ds(k * L, L)]
                for p in range(1, S1):
                    s = s + _rv[j * S1 + p, pl.ds(k * L, L)]
                o = gself_v[_off + j, pl.ds(k * L, L)] + s * (1.0 / S1)
                outb[_off + j, pl.ds(k * L, L)] = jnp.maximum(o, 0.0)
        if ch + NBUF_C < NCH_C:
            cps[ch + NBUF_C] = issue(ch + NBUF_C)
    pltpu.sync_copy(outb, out_hbm.at[pl.ds(base, B_PER_W)])


def _mm3_body(a_ref, p_ref, w1_ref, w2_ref, gs_ref, gn_ref):
    w1 = w1_ref[...]
    h = jnp.dot(a_ref[...], w1[:D], preferred_element_type=jnp.float32)
    h = h + jnp.dot(p_ref[...], w1[D:], preferred_element_type=jnp.float32)
    h = jnp.maximum(h, 0.0)
    w2 = w2_ref[...]
    gs_ref[...] = jnp.dot(h, w2[:OUT], preferred_element_type=jnp.float32)
    gn_ref[...] = jnp.dot(h, w2[OUT:], preferred_element_type=jnp.float32)


def _mm3(a, p, w1, w2, bm):
    m = a.shape[0]
    return pl.pallas_call(
        _mm3_body,
        grid=(m // bm,),
        in_specs=[
            pl.BlockSpec((bm, D), lambda i: (i, 0)),
            pl.BlockSpec((bm, D), lambda i: (i, 0)),
            pl.BlockSpec((2 * D, OUT), lambda i: (0, 0)),
            pl.BlockSpec((2 * OUT, OUT), lambda i: (0, 0)),
        ],
        out_specs=(
            pl.BlockSpec((bm, OUT), lambda i: (i, 0)),
            pl.BlockSpec((bm, OUT), lambda i: (i, 0)),
        ),
        out_shape=(
            jax.ShapeDtypeStruct((m, OUT), jnp.float32),
            jax.ShapeDtypeStruct((m, OUT), jnp.float32),
        ),
    )(a, p, w1, w2)


def kernel(nodes_batch, adj, raw_features, W1, W2):
    idx_a = adj[:, :S2].reshape(-1)
    adj_p = jnp.pad(adj[:, :S1], ((0, 0), (0, 128 - S1)))
    pre_agg = _preagg(idx_a, raw_features)
    gs, gn = _mm3(raw_features, pre_agg, W1, W2, 1000)
    return _batch(nodes_batch, adj_p, gs, gn)


# speedup vs baseline: 1.0145x; 1.0145x over previous
import functools

import jax
import jax.numpy as jnp
from jax import lax
from jax.experimental import pallas as pl
from jax.experimental.pallas import tpu as pltpu
from jax.experimental.pallas import tpu_sc as plsc

N = 10000
DEG = 32
D = 128
OUT = 128
B = 4096
S1 = 16
S2 = 4

NC, NS, L = 2, 16, 16
NW = NC * NS
NODES_PER_W = 320
CH_A = 32
NCH_A = NODES_PER_W // CH_A
NBUF_A = 5
B_PER_W = B // NW
CH_C = 8
NCH_C = B_PER_W // CH_C
NBUF_C = 4

_MESH = plsc.VectorSubcoreMesh(
    core_axis_name="c", subcore_axis_name="s", num_cores=NC, num_subcores=NS
)


@functools.partial(
    pl.kernel,
    out_type=jax.ShapeDtypeStruct((N, D), jnp.float32),
    mesh=_MESH,
    scratch_types=[
        pltpu.VMEM((NODES_PER_W * S2,), jnp.int32),
    ]
    + [pltpu.VMEM((CH_A * S2, D), jnp.float32) for _ in range(NBUF_A)]
    + [pltpu.VMEM((NODES_PER_W, D), jnp.float32)]
    + [pltpu.SemaphoreType.DMA for _ in range(NBUF_A + 1)],
)
def _preagg(idx_hbm, feat_hbm, out_hbm, idx_v, r0, r1, r2, r3, r4, ob,
            s0, s1, s2, s3, s4, so):
    rows = (r0, r1, r2, r3, r4)
    sems = (s0, s1, s2, s3, s4)
    wid = lax.axis_index("s") * NC + lax.axis_index("c")
    base = jnp.minimum(wid * NODES_PER_W, N - NODES_PER_W)
    pltpu.sync_copy(idx_hbm.at[pl.ds(base * S2, NODES_PER_W * S2)], idx_v)

    def issue(ch):
        b = ch % NBUF_A
        src = feat_hbm.at[idx_v.at[pl.ds(ch * CH_A * S2, CH_A * S2)]]
        return pltpu.async_copy(src, rows[b], sems[b])

    cps = {ch: issue(ch) for ch in range(NBUF_A)}
    ocps = []
    for ch in range(NCH_A):
        b = ch % NBUF_A
        cps[ch].wait()

        @plsc.parallel_loop(0, CH_A)
        def node(i, _rv=rows[b], _off=ch * CH_A):
            for k in range(D // L):
                s = _rv[i * S2, pl.ds(k * L, L)]
                for p in range(1, S2):
                    s = s + _rv[i * S2 + p, pl.ds(k * L, L)]
                ob[_off + i, pl.ds(k * L, L)] = s * (1.0 / S2)
        ocps.append(
            pltpu.async_copy(
                ob.at[pl.ds(ch * CH_A, CH_A)],
                out_hbm.at[pl.ds(base + ch * CH_A, CH_A)],
                so,
            )
        )
        if ch + NBUF_A < NCH_A:
            cps[ch + NBUF_A] = issue(ch + NBUF_A)
    for d in ocps:
        d.wait()


@functools.partial(
    pl.kernel,
    out_type=jax.ShapeDtypeStruct((B, OUT), jnp.float32),
    mesh=_MESH,
    scratch_types=[
        pltpu.VMEM((B_PER_W,), jnp.int32),
        pltpu.VMEM((B_PER_W, 128), jnp.int32),
        pltpu.VMEM((B_PER_W, OUT), jnp.float32),
    ]
    + [pltpu.VMEM((CH_C * S1,), jnp.int32) for _ in range(NBUF_C)]
    + [pltpu.VMEM((CH_C * S1, OUT), jnp.float32) for _ in range(NBUF_C)]
    + [pltpu.VMEM((B_PER_W, OUT), jnp.float32)]
    + [pltpu.SemaphoreType.DMA for _ in range(NBUF_C + 3)],
)
def _batch(nb_hbm, adj_hbm, gs_hbm, gn_hbm, out_hbm,
           nb_v, adjr_v, gself_v, i0, i1, i2, i3, r0, r1, r2, r3, outb,
           s0, s1, s2, s3, sa, sh, so):
    nidx = (i0, i1, i2, i3)
    nrows = (r0, r1, r2, r3)
    sems = (s0, s1, s2, s3)
    wid = lax.axis_index("s") * NC + lax.axis_index("c")
    base = wid * B_PER_W
    pltpu.sync_copy(nb_hbm.at[pl.ds(base, B_PER_W)], nb_v)
    cp_adj = pltpu.async_copy(adj_hbm.at[nb_v], adjr_v, sa)
    cp_self = pltpu.async_copy(gs_hbm.at[nb_v], gself_v, sh)
    cp_adj.wait()

    def issue(ch):
        b = ch % NBUF_C
        _ni = nidx[b]

        def build(j, c2, _off=ch * CH_C):
            _ni[pl.ds(j * S1, S1)] = adjr_v[_off + j, pl.ds(0, S1)]
            return c2

        lax.fori_loop(0, CH_C, build, 0)
        return pltpu.async_copy(gn_hbm.at[_ni], nrows[b], sems[b])

    cps = {ch: issue(ch) for ch in range(NBUF_C)}
    cp_self.wait()
    ocps = []
    for ch in range(NCH_C):
        b = ch % NBUF_C
        cps[ch].wait()

        @plsc.parallel_loop(0, CH_C)
        def bacc(j, _rv=nrows[b], _off=ch * CH_C):
            @plsc.parallel_loop(0, OUT // L)
            def kstep(k):
                s = _rv[j * S1, pl.ds(k * L, L)]
                for p in range(1, S1):
                    s = s + _rv[j * S1 + p, pl.ds(k * L, L)]
                o = gself_v[_off + j, pl.ds(k * L, L)] + s * (1.0 / S1)
                outb[_off + j, pl.ds(k * L, L)] = jnp.maximum(o, 0.0)
        ocps.append(
            pltpu.async_copy(
                outb.at[pl.ds(ch * CH_C, CH_C)],
                out_hbm.at[pl.ds(base + ch * CH_C, CH_C)],
                so,
            )
        )
        if ch + NBUF_C < NCH_C:
            cps[ch + NBUF_C] = issue(ch + NBUF_C)
    for d in ocps:
        d.wait()


def _mm3_body(a_ref, p_ref, w1_ref, w2_ref, gs_ref, gn_ref):
    w1 = w1_ref[...]
    h = jnp.dot(a_ref[...], w1[:D], preferred_element_type=jnp.float32)
    h = h + jnp.dot(p_ref[...], w1[D:], preferred_element_type=jnp.float32)
    h = jnp.maximum(h, 0.0)
    w2 = w2_ref[...]
    gs_ref[...] = jnp.dot(h, w2[:OUT], preferred_element_type=jnp.float32)
    gn_ref[...] = jnp.dot(h, w2[OUT:], preferred_element_type=jnp.float32)


def _mm3(a, p, w1, w2, bm):
    m = a.shape[0]
    return pl.pallas_call(
        _mm3_body,
        grid=(m // bm,),
        in_specs=[
            pl.BlockSpec((bm, D), lambda i: (i, 0)),
            pl.BlockSpec((bm, D), lambda i: (i, 0)),
            pl.BlockSpec((2 * D, OUT), lambda i: (0, 0)),
            pl.BlockSpec((2 * OUT, OUT), lambda i: (0, 0)),
        ],
        out_specs=(
            pl.BlockSpec((bm, OUT), lambda i: (i, 0)),
            pl.BlockSpec((bm, OUT), lambda i: (i, 0)),
        ),
        out_shape=(
            jax.ShapeDtypeStruct((m, OUT), jnp.float32),
            jax.ShapeDtypeStruct((m, OUT), jnp.float32),
        ),
    )(a, p, w1, w2)


def kernel(nodes_batch, adj, raw_features, W1, W2):
    idx_a = adj[:, :S2].reshape(-1)
    adj_p = jnp.pad(adj[:, :S1], ((0, 0), (0, 128 - S1)))
    pre_agg = _preagg(idx_a, raw_features)
    gs, gn = _mm3(raw_features, pre_agg, W1, W2, 1000)
    return _batch(nodes_batch, adj_p, gs, gn)
